# baseline (speedup 1.0000x reference)
import functools

import jax
import jax.numpy as jnp
from jax import lax
from jax.experimental import pallas as pl
from jax.experimental.pallas import tpu as pltpu
from jax.experimental.pallas import tpu_sc as plsc
from jax._src.pallas import mpmd
from jax._src.pallas import core as pallas_core

D_BIAS = 64
BATCH = 16384
SPLIT = 7168
CHUNK_I = 512


@functools.lru_cache(maxsize=None)
def _build(batch, d):
    info = plsc.get_sparse_core_info()
    nc = info.num_cores
    nw = nc * info.num_subcores
    s_per_c = SPLIT // nc
    n_chunks = s_per_c // CHUNK_I
    b_tec = batch - SPLIT
    b_per_w = b_tec // nw
    smesh = plsc.ScalarSubcoreMesh(axis_name="c", num_cores=nc)
    vmesh = plsc.VectorSubcoreMesh(core_axis_name="c", subcore_axis_name="s")

    def scs_fn(idx_hbm, table_hbm, out_hbm, idx_sp, stage, idx_v, rows_v):
        cid = lax.axis_index("c")
        base = pl.multiple_of(cid * s_per_c, 8)

        def body(idx_s, gsem):

            def chunk_step(c, _):
                off = c * CHUNK_I
                pltpu.sync_copy(
                    idx_hbm.at[pl.ds(base + off, CHUNK_I)], idx_sp)
                pltpu.sync_copy(idx_sp, idx_s)

                def fire(j, _):
                    u = idx_s[j]
                    pltpu.make_async_copy(
                        table_hbm.at[u], stage.at[off + j], gsem).start()
                    return 0

                lax.fori_loop(0, CHUNK_I, fire, 0)
                return 0

            lax.fori_loop(0, n_chunks, chunk_step, 0)

            def drain(j, _):
                pltpu.make_async_copy(
                    table_hbm.at[0], stage.at[j], gsem).wait()
                return 0

            lax.fori_loop(0, s_per_c, drain, 0)
            pltpu.sync_copy(stage, out_hbm.at[pl.ds(base, s_per_c)])

        pl.run_scoped(
            body,
            pltpu.SMEM((CHUNK_I,), jnp.int32),
            pltpu.SemaphoreType.DMA,
        )

    def tec_fn(idx_hbm, table_hbm, out_hbm, idx_sp, stage, idx_v, rows_v):
        wid = lax.axis_index("s") * nc + lax.axis_index("c")
        base = pl.multiple_of(SPLIT + wid * b_per_w, 8)

        def body(vsem):
            pltpu.sync_copy(idx_hbm.at[pl.ds(base, b_per_w)], idx_v)

            def fire(g, _):
                u16 = idx_v[pl.ds(g * 16, 16)]
                for j in range(16):
                    u = u16[j]
                    pltpu.make_async_copy(
                        table_hbm.at[u], rows_v.at[g * 16 + j], vsem
                    ).start()
                return 0

            lax.fori_loop(0, b_per_w // 16, fire, 0)

            def drain(g, _):
                for j in range(16):
                    pltpu.make_async_copy(
                        table_hbm.at[0], rows_v.at[g * 16 + j], vsem
                    ).wait()
                return 0

            lax.fori_loop(0, b_per_w // 16, drain, 0)
            pltpu.sync_copy(rows_v, out_hbm.at[pl.ds(base, b_per_w)])

        pl.run_scoped(body, pltpu.SemaphoreType.DMA)

    fn = mpmd.mpmd_map(
        [(vmesh, tec_fn), (smesh, scs_fn)],
        out_types=jax.ShapeDtypeStruct((batch, d), jnp.float32),
        scratch_types=[
            pltpu.VMEM_SHARED((CHUNK_I,), jnp.int32),
            pltpu.VMEM_SHARED((s_per_c, d), jnp.float32),
            pallas_core.CoreMemorySpace(pltpu.VMEM, vmesh)(
                (b_per_w,), jnp.int32),
            pallas_core.CoreMemorySpace(pltpu.VMEM, vmesh)(
                (b_per_w, d), jnp.float32),
        ],
    )
    return fn


def kernel(user_id, mu_embed, log_var_embed):
    del log_var_embed
    return _build(BATCH, D_BIAS)(user_id.astype(jnp.int32), mu_embed)


# speedup vs baseline: 1.0075x; 1.0075x over previous
import functools

import jax
import jax.numpy as jnp
from jax import lax
from jax.experimental import pallas as pl
from jax.experimental.pallas import tpu as pltpu
from jax.experimental.pallas import tpu_sc as plsc

D_BIAS = 64
BATCH = 16384


@functools.lru_cache(maxsize=None)
def _build_gather(batch, d):
    info = plsc.get_sparse_core_info()
    nw = info.num_cores * info.num_subcores
    b_per_w = batch // nw
    mesh = plsc.VectorSubcoreMesh(core_axis_name="c", subcore_axis_name="s")

    @functools.partial(
        pl.kernel,
        mesh=mesh,
        out_type=jax.ShapeDtypeStruct((batch, d), jnp.float32),
        scratch_types=[
            pltpu.VMEM((b_per_w,), jnp.int32),
            pltpu.VMEM((b_per_w, d), jnp.float32),
            pltpu.SemaphoreType.DMA((16,)),
        ],
    )
    def gather(idx_hbm, table_hbm, out_hbm, idx_v, rows_v, sem):
        wid = lax.axis_index("s") * info.num_cores + lax.axis_index("c")
        base = wid * b_per_w
        pltpu.sync_copy(idx_hbm.at[pl.ds(base, b_per_w)], idx_v)

        def fire(g, _):
            u16 = idx_v[pl.ds(g * 16, 16)]
            for j in range(16):
                u = u16[j]
                pltpu.make_async_copy(
                    table_hbm.at[u], rows_v.at[g * 16 + j], sem.at[j]
                ).start()
            return 0

        lax.fori_loop(0, b_per_w // 16, fire, 0)

        def drain(g, _):
            for j in range(16):
                pltpu.make_async_copy(
                    table_hbm.at[0], rows_v.at[g * 16 + j], sem.at[j]
                ).wait()
            return 0

        lax.fori_loop(0, b_per_w // 16, drain, 0)
        pltpu.sync_copy(rows_v, out_hbm.at[pl.ds(base, b_per_w)])

    return gather


def kernel(user_id, mu_embed, log_var_embed):
    del log_var_embed
    return _build_gather(BATCH, D_BIAS)(user_id.astype(jnp.int32), mu_embed)


# speedup vs baseline: 1.0907x; 1.0826x over previous
import functools

import jax
import jax.numpy as jnp
from jax import lax
from jax.experimental import pallas as pl
from jax.experimental.pallas import tpu as pltpu
from jax.experimental.pallas import tpu_sc as plsc

D_BIAS = 64
BATCH = 16384


@functools.lru_cache(maxsize=None)
def _build_gather(batch, d):
    info = plsc.get_sparse_core_info()
    nw = info.num_cores * info.num_subcores
    b_per_w = batch // nw
    mesh = plsc.VectorSubcoreMesh(core_axis_name="c", subcore_axis_name="s")

    @functools.partial(
        pl.kernel,
        mesh=mesh,
        out_type=jax.ShapeDtypeStruct((batch, d), jnp.float32),
        scratch_types=[
            pltpu.VMEM((b_per_w,), jnp.int32),
            pltpu.VMEM((b_per_w, d), jnp.float32),
            pltpu.SemaphoreType.DMA,
        ],
    )
    def gather(idx_hbm, table_hbm, out_hbm, idx_v, rows_v, sem):
        wid = lax.axis_index("s") * info.num_cores + lax.axis_index("c")
        base = wid * b_per_w
        pltpu.sync_copy(idx_hbm.at[pl.ds(base, b_per_w)], idx_v)

        def fire(g, _):
            u16 = idx_v[pl.ds(g * 16, 16)]
            for j in range(16):
                u = u16[j]
                pltpu.make_async_copy(
                    table_hbm.at[u], rows_v.at[g * 16 + j], sem
                ).start()
            return 0

        lax.fori_loop(0, b_per_w // 16, fire, 0)

        def drain(g, _):
            for j in range(16):
                pltpu.make_async_copy(
                    table_hbm.at[0], rows_v.at[g * 16 + j], sem
                ).wait()
            return 0

        lax.fori_loop(0, b_per_w // 16, drain, 0)
        pltpu.sync_copy(rows_v, out_hbm.at[pl.ds(base, b_per_w)])

    return gather


def kernel(user_id, mu_embed, log_var_embed):
    del log_var_embed
    return _build_gather(BATCH, D_BIAS)(user_id.astype(jnp.int32), mu_embed)
